# Initial kernel scaffold; baseline (speedup 1.0000x reference)
import jax
import jax.numpy as jnp
from jax.experimental import pallas as pl


def kernel(img, hand, LEGHTN, wconv, wfc, bfc, w1cat, b1cat, w2, b2):
    raise NotImplementedError("write your pallas kernel here")


# speedup vs baseline: 23.4519x; 23.4519x over previous
import functools

import jax
import jax.numpy as jnp
from jax.experimental import pallas as pl
from jax.experimental.pallas import tpu as pltpu

KSIZE, STRIDE, PAD = 7, 2, 3
VMEM_LIMIT = 48 * 1024 * 1024


def _round_up(x, m):
    return -(-x // m) * m


def _backbone_kernel(x_ref, wt_ref, out_ref, a_ref, *, ho, wo):
    C = x_ref.shape[3]
    ktrue = C * KSIZE * KSIZE
    kp = a_ref.shape[0]
    if kp > ktrue:
        a_ref[ktrue:, :, :] = jnp.zeros((kp - ktrue,) + a_ref.shape[1:],
                                        a_ref.dtype)
    for c in range(C):
        for i in range(KSIZE):
            rp, p = i % 2, i // 2
            for j in range(KSIZE):
                cp, s = j % 2, j // 2
                k = c * KSIZE * KSIZE + i * KSIZE + j
                blk = x_ref[0, rp, cp, c, p:p + ho, :]
                if s:
                    blk = jnp.concatenate([blk[:, s:], blk[:, :s]], axis=1)
                a_ref[k, :, :] = blk
    z = jnp.einsum('ck,kpm->cpm', wt_ref[...], a_ref[...],
                   preferred_element_type=jnp.float32)
    z = jnp.maximum(z, 0.0)
    lane = jax.lax.broadcasted_iota(jnp.int32, z.shape, 2)
    z = jnp.where(lane < wo, z, 0.0)
    out_ref[...] = jnp.sum(z, axis=(1, 2)).reshape(1, 1, -1)


def _heads_kernel(pool_ref, wfc_ref, bfc_ref, w1_ref, b1_ref, w2_ref, b2_ref,
                  out_ref, *, inv_p):
    pooled = (pool_ref[...] * inv_p).astype(jnp.bfloat16)
    feat = (jnp.dot(pooled, wfc_ref[...],
                    preferred_element_type=jnp.float32) + bfc_ref[...])
    x = feat.astype(jnp.bfloat16)
    h = (jnp.dot(x, w1_ref[...],
                 preferred_element_type=jnp.float32) + b1_ref[...])
    h = jnp.maximum(h, 0.0).astype(jnp.bfloat16)
    out_ref[0] = (jnp.dot(h, w2_ref[0],
                          preferred_element_type=jnp.float32) + b2_ref[0])


def kernel(img, hand, LEGHTN, wconv, wfc, bfc, w1cat, b1cat, w2, b2):
    x = img[:, -1].astype(jnp.bfloat16)
    B, C, H, W = x.shape
    HO = (H + 2 * PAD - KSIZE) // STRIDE + 1
    WO = (W + 2 * PAD - KSIZE) // STRIDE + 1
    C1 = wconv.shape[1]
    KP = _round_up(C * KSIZE * KSIZE, 16)
    MP = _round_up(HO + KSIZE // 2, 16)
    L = _round_up(WO + KSIZE // 2, 128)

    xpb = jnp.pad(x, ((0, 0), (0, 0),
                      (PAD, 2 * MP - H - PAD), (PAD, 2 * L - W - PAD)))
    xpp = xpb.reshape(B, C, MP, 2, L, 2).transpose(0, 3, 5, 1, 2, 4)
    wt = wconv[:KP].T

    pooled = pl.pallas_call(
        functools.partial(_backbone_kernel, ho=HO, wo=WO),
        out_shape=jax.ShapeDtypeStruct((B, 1, C1), jnp.float32),
        grid=(B,),
        in_specs=[
            pl.BlockSpec((1, 2, 2, C, MP, L), lambda b: (b, 0, 0, 0, 0, 0)),
            pl.BlockSpec((C1, KP), lambda b: (0, 0)),
        ],
        out_specs=pl.BlockSpec((1, 1, C1), lambda b: (b, 0, 0)),
        scratch_shapes=[pltpu.VMEM((KP, HO, L), jnp.bfloat16)],
        compiler_params=pltpu.CompilerParams(
            dimension_semantics=("parallel",),
            vmem_limit_bytes=VMEM_LIMIT),
    )(xpp, wt)
    pooled = pooled.reshape(B, C1)

    D = wfc.shape[1]
    NH, Hd, O = w2.shape
    heads = pl.pallas_call(
        functools.partial(_heads_kernel, inv_p=1.0 / float(HO * WO)),
        out_shape=jax.ShapeDtypeStruct((NH, B, O), jnp.float32),
        grid=(NH,),
        in_specs=[
            pl.BlockSpec((B, C1), lambda h: (0, 0)),
            pl.BlockSpec((C1, D), lambda h: (0, 0)),
            pl.BlockSpec((1, D), lambda h: (0, 0)),
            pl.BlockSpec((D, Hd), lambda h: (0, h)),
            pl.BlockSpec((1, Hd), lambda h: (0, h)),
            pl.BlockSpec((1, Hd, O), lambda h: (h, 0, 0)),
            pl.BlockSpec((1, 1, O), lambda h: (h, 0, 0)),
        ],
        out_specs=pl.BlockSpec((1, B, O), lambda h: (h, 0, 0)),
        compiler_params=pltpu.CompilerParams(
            dimension_semantics=("parallel",),
            vmem_limit_bytes=VMEM_LIMIT),
    )(pooled, wfc, bfc, w1cat, b1cat, w2, b2)

    return [jnp.transpose(heads, (1, 0, 2))]
